# baseline (speedup 1.0000x reference)
import functools

import jax
import jax.numpy as jnp
from jax import lax
from jax.experimental import pallas as pl
from jax.experimental.pallas import tpu as pltpu
from jax.experimental.pallas import tpu_sc as plsc

VOCAB = 1000
DIM = 1000
DIMP = 1024
BATCH = 4096
BLOCK = 8
NT = DIMP // 128
NB = BATCH * BLOCK
NW = 32
NSLICE = 4
SPS = BLOCK // NSLICE
RS = SPS * BATCH
B_PER_W = RS // NW
CHUNK = 32
NCHUNK = B_PER_W // CHUNK
NBUF = 2


def _sc_gather(idx, table3):
    mesh = plsc.VectorSubcoreMesh(core_axis_name="c", subcore_axis_name="s")

    @functools.partial(
        pl.kernel,
        mesh=mesh,
        compiler_params=pltpu.CompilerParams(use_tc_tiling_on_sc=False),
        out_type=jax.ShapeDtypeStruct((NT, RS, 128), jnp.float32),
        scratch_types=[
            pltpu.VMEM((NCHUNK, CHUNK), jnp.int32),
        ]
        + [pltpu.VMEM((CHUNK, NT, 128), jnp.float32) for _ in range(NBUF)]
        + [pltpu.SemaphoreType.DMA for _ in range(2 * NBUF)],
    )
    def k(idx_hbm, table_hbm, out_hbm, idx_v, *bufs_sems):
        bufs = bufs_sems[:NBUF]
        gsems = bufs_sems[NBUF : 2 * NBUF]
        wsems = bufs_sems[2 * NBUF :]
        wid = lax.axis_index("s") * 2 + lax.axis_index("c")
        pltpu.sync_copy(idx_hbm.at[wid], idx_v)
        base = wid * B_PER_W

        def start_gather(g):
            b = g % NBUF
            return pltpu.async_copy(table_hbm.at[idx_v.at[g]], bufs[b], gsems[b])

        def start_write(g):
            b = g % NBUF
            return [
                pltpu.async_copy(
                    bufs[b].at[:, t],
                    out_hbm.at[t].at[pl.ds(base + g * CHUNK, CHUNK)],
                    wsems[b],
                )
                for t in range(NT)
            ]

        gathers = [None] * NBUF
        writes = [None] * NBUF
        gathers[0] = start_gather(0)
        for g in range(NCHUNK):
            b = g % NBUF
            gathers[b].wait()
            writes[b] = start_write(g)
            if g + 1 < NCHUNK:
                b2 = (g + 1) % NBUF
                if writes[b2] is not None:
                    for w in writes[b2]:
                        w.wait()
                gathers[b2] = start_gather(g + 1)
        for ws in writes:
            if ws is not None:
                for w in ws:
                    w.wait()

    return k(idx, table3)


def _tc_finish(y, z_prev, slice_idx):
    def body(y_ref, zp_ref, z_ref):
        del zp_ref
        v = y_ref[...].reshape(BATCH // 128, 128, 16, BLOCK)
        z_ref[...] = jnp.transpose(v, (2, 0, 3, 1))[None]

    kwargs = {}
    operands = [y]
    in_specs = [pl.BlockSpec((1, BATCH, 128), lambda t, s: (t, s, 0))]
    if z_prev is None:
        def body0(y_ref, z_ref):
            v = y_ref[...].reshape(BATCH // 128, 128, 16, BLOCK)
            z_ref[...] = jnp.transpose(v, (2, 0, 3, 1))[None]
        fn = body0
    else:
        fn = body
        operands.append(z_prev)
        in_specs.append(pl.BlockSpec(memory_space=pl.ANY))
        kwargs["input_output_aliases"] = {1: 0}

    return pl.pallas_call(
        fn,
        grid=(NT, SPS),
        in_specs=in_specs,
        out_specs=pl.BlockSpec(
            (1, 16, BATCH // 128, BLOCK, 128),
            lambda t, s, _k=slice_idx: (_k * SPS + s, t, 0, 0, 0),
        ),
        out_shape=jax.ShapeDtypeStruct(
            (BLOCK, DIM // BLOCK, BATCH // 128, BLOCK, 128), jnp.float32
        ),
        **kwargs,
    )(*operands)


def kernel(inputs, table):
    idx = inputs.astype(jnp.int32).T.reshape(NSLICE, NW, NCHUNK, CHUNK)
    table3 = jnp.pad(table, ((0, 0), (0, DIMP - DIM))).reshape(VOCAB, NT, 128)
    z = None
    for k in range(NSLICE):
        y = _sc_gather(idx[k], table3)
        z = _tc_finish(y, z, k)
    return jnp.transpose(z, (2, 4, 0, 1, 3)).reshape(BATCH, BLOCK, DIM)


# speedup vs baseline: 6.1762x; 6.1762x over previous
import functools

import jax
import jax.numpy as jnp
from jax import lax
from jax.experimental import pallas as pl
from jax.experimental.pallas import tpu as pltpu
from jax.experimental.pallas import tpu_sc as plsc

VOCAB = 1000
DIM = 1000
DIMP = 1024
BATCH = 4096
BLOCK = 8
NT = DIMP // 128
NB = BATCH * BLOCK
NW = 32
NSLICE = 4
SPS = BLOCK // NSLICE
RS = SPS * BATCH
B_PER_W = RS // NW
CHUNK = 32
NCHUNK = B_PER_W // CHUNK
NBUF = 2


def _sc_gather(idx, table3):
    mesh = plsc.VectorSubcoreMesh(core_axis_name="c", subcore_axis_name="s")

    @functools.partial(
        pl.kernel,
        mesh=mesh,
        compiler_params=pltpu.CompilerParams(use_tc_tiling_on_sc=False),
        out_type=jax.ShapeDtypeStruct((NT, RS, 128), jnp.float32),
        scratch_types=[
            pltpu.VMEM((NCHUNK, CHUNK), jnp.int32),
        ]
        + [pltpu.VMEM((CHUNK, NT, 128), jnp.float32) for _ in range(NBUF)]
        + [pltpu.SemaphoreType.DMA for _ in range(2 * NBUF)],
    )
    def k(idx_hbm, table_hbm, out_hbm, idx_v, *bufs_sems):
        bufs = bufs_sems[:NBUF]
        gsems = bufs_sems[NBUF : 2 * NBUF]
        wsems = bufs_sems[2 * NBUF :]
        wid = lax.axis_index("s") * 2 + lax.axis_index("c")
        pltpu.sync_copy(idx_hbm.at[wid], idx_v)
        base = wid * B_PER_W

        def start_gather(g):
            b = g % NBUF
            return pltpu.async_copy(table_hbm.at[idx_v.at[g]], bufs[b], gsems[b])

        def start_write(g):
            b = g % NBUF
            return [
                pltpu.async_copy(
                    bufs[b].at[:, t],
                    out_hbm.at[t].at[pl.ds(base + g * CHUNK, CHUNK)],
                    wsems[b],
                )
                for t in range(NT)
            ]

        gathers = [None] * NBUF
        writes = [None] * NBUF
        gathers[0] = start_gather(0)
        for g in range(NCHUNK):
            b = g % NBUF
            gathers[b].wait()
            writes[b] = start_write(g)
            if g + 1 < NCHUNK:
                b2 = (g + 1) % NBUF
                if writes[b2] is not None:
                    for w in writes[b2]:
                        w.wait()
                gathers[b2] = start_gather(g + 1)
        for ws in writes:
            if ws is not None:
                for w in ws:
                    w.wait()

    return k(idx, table3)


def _tc_finish(y, z_prev, slice_idx):
    def body(y_ref, zp_ref, z_ref):
        del zp_ref
        v = y_ref[...].reshape(BATCH, 128)
        z_ref[...] = jnp.transpose(v, (1, 0)).reshape(1, 128, BATCH)

    kwargs = {}
    operands = [y]
    in_specs = [pl.BlockSpec((1, BATCH, 128), lambda t, s: (t, s, 0))]
    if z_prev is None:
        def body0(y_ref, z_ref):
            v = y_ref[...].reshape(BATCH, 128)
            z_ref[...] = jnp.transpose(v, (1, 0)).reshape(1, 128, BATCH)
        fn = body0
    else:
        fn = body
        operands.append(z_prev)
        in_specs.append(pl.BlockSpec(memory_space=pl.ANY))
        kwargs["input_output_aliases"] = {1: 0}

    return pl.pallas_call(
        fn,
        grid=(NT, SPS),
        in_specs=in_specs,
        out_specs=pl.BlockSpec(
            (1, 128, BATCH),
            lambda t, s, _k=slice_idx: (_k * SPS + s, t, 0),
        ),
        out_shape=jax.ShapeDtypeStruct((BLOCK, DIM, BATCH), jnp.float32),
        **kwargs,
    )(*operands)


def kernel(inputs, table):
    idx = inputs.astype(jnp.int32).T.reshape(NSLICE, NW, NCHUNK, CHUNK)
    table3 = jnp.pad(table, ((0, 0), (0, DIMP - DIM))).reshape(VOCAB, NT, 128)
    z = None
    for k in range(NSLICE):
        y = _sc_gather(idx[k], table3)
        z = _tc_finish(y, z, k)
    return jnp.transpose(z, (2, 0, 1))
